# Initial kernel scaffold; baseline (speedup 1.0000x reference)
import jax
import jax.numpy as jnp
from jax.experimental import pallas as pl


def kernel(features, edge_index, batch_nodes, device, W1, b1, k2_1, W2, b2, k2_2, g1, be1, g2, be2):
    raise NotImplementedError("write your pallas kernel here")


# speedup vs baseline: 7.5851x; 7.5851x over previous
import functools

import jax
import jax.numpy as jnp
from jax import lax
from jax.experimental import pallas as pl
from jax.experimental.pallas import tpu as pltpu
from jax.experimental.pallas import tpu_sc as plsc

N = 10000
E = 320000
F = 128
B = 1024

NC, NS = 2, 16
NW = NC * NS
NPAD = 10112
RPT = NPAD // NS

CHUNK = 128
NCHUNK = 80
EPW = NCHUNK * CHUNK
EPAD = EPW * NW
ECHUNKS = EPAD // CHUNK

BLK = 1000
GRID = N // BLK

_MESH = plsc.VectorSubcoreMesh(
    core_axis_name="c", subcore_axis_name="s", num_cores=NC, num_subcores=NS)


def _deg_body(dst2d, zerosF, onesF, out, table, ones_v, idx_v, sem):
    c = lax.axis_index("c")
    s = lax.axis_index("s")
    wid = c * NS + s
    pltpu.sync_copy(zerosF.at[pl.ds(s * RPT, RPT)], table.at[pl.ds(s * RPT, RPT)])
    pltpu.sync_copy(onesF, ones_v)
    pltpu.sync_copy(dst2d.at[pl.ds(wid * NCHUNK, NCHUNK)], idx_v)
    plsc.subcore_barrier()

    def chunk(j, carry):
        pltpu.sync_copy(ones_v, table.at[idx_v.at[j]], add=True)
        return carry

    lax.fori_loop(0, NCHUNK, chunk, 0)
    plsc.subcore_barrier()
    pltpu.sync_copy(table.at[pl.ds(s * RPT, RPT)],
                    out.at[pl.ds(c * NPAD + s * RPT, RPT)])


_deg_call = functools.partial(
    pl.kernel,
    out_type=jax.ShapeDtypeStruct((NC * NPAD, F), jnp.float32),
    mesh=_MESH,
    scratch_types=[
        pltpu.VMEM_SHARED((NPAD, F), jnp.float32),
        pltpu.VMEM((CHUNK, F), jnp.float32),
        pltpu.VMEM((NCHUNK, CHUNK), jnp.int32),
        pltpu.SemaphoreType.DMA,
    ],
)(_deg_body)


def _edge_body(hs, src2d, dst2d, zerosF, out, agg, src_v, dst_v, rows, sem):
    c = lax.axis_index("c")
    s = lax.axis_index("s")
    wid = c * NS + s
    pltpu.sync_copy(zerosF.at[pl.ds(s * RPT, RPT)], agg.at[pl.ds(s * RPT, RPT)])
    pltpu.sync_copy(src2d.at[pl.ds(wid * NCHUNK, NCHUNK)], src_v)
    pltpu.sync_copy(dst2d.at[pl.ds(wid * NCHUNK, NCHUNK)], dst_v)
    plsc.subcore_barrier()

    def chunk(j, carry):
        pltpu.async_copy(hs.at[src_v.at[j]], rows, sem).wait()
        pltpu.sync_copy(rows, agg.at[dst_v.at[j]], add=True)
        return carry

    lax.fori_loop(0, NCHUNK, chunk, 0)
    plsc.subcore_barrier()
    pltpu.sync_copy(agg.at[pl.ds(s * RPT, RPT)],
                    out.at[pl.ds(c * NPAD + s * RPT, RPT)])


_edge_call = functools.partial(
    pl.kernel,
    out_type=jax.ShapeDtypeStruct((NC * NPAD, F), jnp.float32),
    mesh=_MESH,
    scratch_types=[
        pltpu.VMEM_SHARED((NPAD, F), jnp.float32),
        pltpu.VMEM((NCHUNK, CHUNK), jnp.int32),
        pltpu.VMEM((NCHUNK, CHUNK), jnp.int32),
        pltpu.VMEM((CHUNK, F), jnp.float32),
        pltpu.SemaphoreType.DMA,
    ],
)(_edge_body)


_BPW = B // NW


def _gather_body(y2, idx_hbm, out, idx_v, rows, sem):
    c = lax.axis_index("c")
    s = lax.axis_index("s")
    wid = c * NS + s
    base = wid * _BPW
    pltpu.sync_copy(idx_hbm.at[pl.ds(base, _BPW)], idx_v)
    pltpu.async_copy(y2.at[idx_v], rows, sem).wait()
    pltpu.sync_copy(rows, out.at[pl.ds(base, _BPW)])


_gather_call = functools.partial(
    pl.kernel,
    out_type=jax.ShapeDtypeStruct((B, F), jnp.float32),
    mesh=_MESH,
    scratch_types=[
        pltpu.VMEM((_BPW,), jnp.int32),
        pltpu.VMEM((_BPW, F), jnp.float32),
        pltpu.SemaphoreType.DMA,
    ],
)(_gather_body)


def _mm1_body(x, w, d0, d1, h_o, hs_o, dinv_o):
    h = jnp.dot(x[...], w[...], preferred_element_type=jnp.float32,
                precision=lax.Precision.HIGHEST)
    deg = d0[...] + d1[...] + 1.0
    dinv = lax.rsqrt(jnp.maximum(deg, 1.0))
    h_o[...] = h
    hs_o[...] = dinv * h
    dinv_o[...] = dinv


_mm1_call = pl.pallas_call(
    _mm1_body,
    grid=(GRID,),
    in_specs=[
        pl.BlockSpec((BLK, F), lambda i: (i, 0)),
        pl.BlockSpec((F, F), lambda i: (0, 0)),
        pl.BlockSpec((BLK, 1), lambda i: (i, 0)),
        pl.BlockSpec((BLK, 1), lambda i: (i, 0)),
    ],
    out_specs=[
        pl.BlockSpec((BLK, F), lambda i: (i, 0)),
        pl.BlockSpec((BLK, F), lambda i: (i, 0)),
        pl.BlockSpec((BLK, 1), lambda i: (i, 0)),
    ],
    out_shape=[
        jax.ShapeDtypeStruct((N, F), jnp.float32),
        jax.ShapeDtypeStruct((N, F), jnp.float32),
        jax.ShapeDtypeStruct((N, 1), jnp.float32),
    ],
)


def _mm2_body(x, w, dinv, h_o, hs_o):
    h = jnp.dot(x[...], w[...], preferred_element_type=jnp.float32,
                precision=lax.Precision.HIGHEST)
    h_o[...] = h
    hs_o[...] = dinv[...] * h


_mm2_call = pl.pallas_call(
    _mm2_body,
    grid=(GRID,),
    in_specs=[
        pl.BlockSpec((BLK, F), lambda i: (i, 0)),
        pl.BlockSpec((F, F), lambda i: (0, 0)),
        pl.BlockSpec((BLK, 1), lambda i: (i, 0)),
    ],
    out_specs=[
        pl.BlockSpec((BLK, F), lambda i: (i, 0)),
        pl.BlockSpec((BLK, F), lambda i: (i, 0)),
    ],
    out_shape=[
        jax.ShapeDtypeStruct((N, F), jnp.float32),
        jax.ShapeDtypeStruct((N, F), jnp.float32),
    ],
)


def _stats_body(s0, s1, h, dinv, k2, b, y_o, st_o, acc):
    i = pl.program_id(0)

    @pl.when(i == 0)
    def _init():
        acc[...] = jnp.zeros_like(acc)

    hv = h[...]
    dv = dinv[...]
    agg = dv * (s0[...] + s1[...] + dv * hv)
    k2v = k2[0, 0]
    y = agg - k2v * hv + b[...]
    y_o[...] = y
    r = hv - agg - k2v * hv
    acc[0:1, :] += jnp.sum(y, axis=0, keepdims=True)
    acc[1:2, :] += jnp.sum(y * y, axis=0, keepdims=True)
    acc[2:3, :] += jnp.sum(r * r, axis=0, keepdims=True)

    @pl.when(i == GRID - 1)
    def _fin():
        a = acc[...]
        loss = jnp.sum(a[2:3, :]) * (1.0 / (N * F))
        st_o[...] = jnp.concatenate(
            [a[0:3, :], jnp.full((1, F), loss, jnp.float32),
             jnp.zeros((4, F), jnp.float32)], axis=0)


_stats_call = pl.pallas_call(
    _stats_body,
    grid=(GRID,),
    in_specs=[
        pl.BlockSpec((BLK, F), lambda i: (i, 0)),
        pl.BlockSpec((BLK, F), lambda i: (i, 0)),
        pl.BlockSpec((BLK, F), lambda i: (i, 0)),
        pl.BlockSpec((BLK, 1), lambda i: (i, 0)),
        pl.BlockSpec((1, 1), lambda i: (0, 0)),
        pl.BlockSpec((1, F), lambda i: (0, 0)),
    ],
    out_specs=[
        pl.BlockSpec((BLK, F), lambda i: (i, 0)),
        pl.BlockSpec((8, F), lambda i: (0, 0)),
    ],
    out_shape=[
        jax.ShapeDtypeStruct((N, F), jnp.float32),
        jax.ShapeDtypeStruct((8, F), jnp.float32),
    ],
    scratch_shapes=[pltpu.VMEM((8, F), jnp.float32)],
)


def _bn_coeffs(st, g, be):
    mu = st[0:1, :] * (1.0 / N)
    var = st[1:2, :] * (1.0 / N) - mu * mu
    inv = lax.rsqrt(var + 1e-5)
    return mu, inv * g[...], be[...]


def _apply_body(y, st, g, be, x_o):
    mu, scale, shift = _bn_coeffs(st, g, be)
    x_o[...] = jnp.tanh((y[...] - mu) * scale + shift)


_apply_call = pl.pallas_call(
    _apply_body,
    grid=(GRID,),
    in_specs=[
        pl.BlockSpec((BLK, F), lambda i: (i, 0)),
        pl.BlockSpec((8, F), lambda i: (0, 0)),
        pl.BlockSpec((1, F), lambda i: (0, 0)),
        pl.BlockSpec((1, F), lambda i: (0, 0)),
    ],
    out_specs=pl.BlockSpec((BLK, F), lambda i: (i, 0)),
    out_shape=jax.ShapeDtypeStruct((N, F), jnp.float32),
)


def _final_body(yb, st, g, be, o):
    mu, scale, shift = _bn_coeffs(st, g, be)
    t = jnp.tanh((yb[...] - mu) * scale + shift)
    m = jnp.max(t, axis=1, keepdims=True)
    lse = jnp.log(jnp.sum(jnp.exp(t - m), axis=1, keepdims=True)) + m
    o[...] = t - lse


_final_call = pl.pallas_call(
    _final_body,
    grid=(1,),
    in_specs=[
        pl.BlockSpec((B, F), lambda i: (0, 0)),
        pl.BlockSpec((8, F), lambda i: (0, 0)),
        pl.BlockSpec((1, F), lambda i: (0, 0)),
        pl.BlockSpec((1, F), lambda i: (0, 0)),
    ],
    out_specs=pl.BlockSpec((B, F), lambda i: (0, 0)),
    out_shape=jax.ShapeDtypeStruct((B, F), jnp.float32),
)


def kernel(features, edge_index, batch_nodes, device, W1, b1, k2_1, W2, b2,
           k2_2, g1, be1, g2, be2):
    del device
    src = edge_index[0].astype(jnp.int32)
    dst = edge_index[1].astype(jnp.int32)
    npad = EPAD - E
    src2d = jnp.concatenate([src, jnp.zeros((npad,), jnp.int32)]).reshape(
        ECHUNKS, CHUNK)
    dst2d = jnp.concatenate([dst, jnp.full((npad,), N, jnp.int32)]).reshape(
        ECHUNKS, CHUNK)
    zerosF = jnp.zeros((NPAD, F), jnp.float32)
    onesF = jnp.ones((CHUNK, F), jnp.float32)

    deg_part = _deg_call(dst2d, zerosF, onesF)
    deg0 = deg_part[0:N, 0:1]
    deg1 = deg_part[NPAD:NPAD + N, 0:1]

    h1, hs1, dinv = _mm1_call(features, W1, deg0, deg1)
    s1p = _edge_call(hs1, src2d, dst2d, zerosF)
    y1, st1 = _stats_call(s1p[0:N], s1p[NPAD:NPAD + N], h1, dinv,
                          k2_1.reshape(1, 1), b1.reshape(1, F))
    x2 = _apply_call(y1, st1, g1.reshape(1, F), be1.reshape(1, F))

    h2, hs2 = _mm2_call(x2, W2, dinv)
    s2p = _edge_call(hs2, src2d, dst2d, zerosF)
    y2, st2 = _stats_call(s2p[0:N], s2p[NPAD:NPAD + N], h2, dinv,
                          k2_2.reshape(1, 1), b2.reshape(1, F))

    yb = _gather_call(y2, batch_nodes.astype(jnp.int32))
    logp = _final_call(yb, st2, g2.reshape(1, F), be2.reshape(1, F))
    return logp, st1[3, 0]


# speedup vs baseline: 8.4066x; 1.1083x over previous
import functools

import jax
import jax.numpy as jnp
from jax import lax
from jax.experimental import pallas as pl
from jax.experimental.pallas import tpu as pltpu
from jax.experimental.pallas import tpu_sc as plsc

N = 10000
E = 320000
F = 128
B = 1024

NC, NS = 2, 16
NW = NC * NS
NPAD = 10112
RPT = NPAD // NS

CHUNK = 128
NCHUNK = 80
EPW = NCHUNK * CHUNK
EPAD = EPW * NW
ECHUNKS = EPAD // CHUNK

BLK = 1000
GRID = N // BLK

_MESH = plsc.VectorSubcoreMesh(
    core_axis_name="c", subcore_axis_name="s", num_cores=NC, num_subcores=NS)


_DEG_GRP = 8


def _deg_body(dst2d, zerosF, onesF, out, table, ones_v, idx_v, sem):
    c = lax.axis_index("c")
    s = lax.axis_index("s")
    wid = c * NS + s
    pltpu.sync_copy(zerosF.at[pl.ds(s * RPT, RPT)], table.at[pl.ds(s * RPT, RPT)])
    pltpu.sync_copy(onesF, ones_v)
    pltpu.sync_copy(dst2d.at[pl.ds(wid * NCHUNK, NCHUNK)], idx_v)
    plsc.subcore_barrier()

    def group(g, carry):
        for b in range(_DEG_GRP):
            pltpu.async_copy(ones_v, table.at[idx_v.at[g * _DEG_GRP + b]], sem,
                             add=True)
        for b in range(_DEG_GRP):
            pltpu.make_async_copy(ones_v, table.at[idx_v.at[g * _DEG_GRP + b]],
                                  sem).wait()
        return carry

    lax.fori_loop(0, NCHUNK // _DEG_GRP, group, 0)
    plsc.subcore_barrier()
    pltpu.sync_copy(table.at[pl.ds(s * RPT, RPT)],
                    out.at[pl.ds(c * NPAD + s * RPT, RPT)])


_deg_call = functools.partial(
    pl.kernel,
    out_type=jax.ShapeDtypeStruct((NC * NPAD, F), jnp.float32),
    mesh=_MESH,
    scratch_types=[
        pltpu.VMEM_SHARED((NPAD, F), jnp.float32),
        pltpu.VMEM((CHUNK, F), jnp.float32),
        pltpu.VMEM((NCHUNK, CHUNK), jnp.int32),
        pltpu.SemaphoreType.DMA,
    ],
)(_deg_body)


GI = 16
NG = NCHUNK // GI


def _edge_body(hs, src2d, dst2d, zerosF, out, agg,
               src_a, dst_a, src_b, dst_b, rows0, rows1, gsem0, gsem1, isem):
    c = lax.axis_index("c")
    s = lax.axis_index("s")
    wid = c * NS + s
    base = wid * NCHUNK
    idx = [(src_a, dst_a), (src_b, dst_b)]
    rows = [rows0, rows1]
    gsems = [gsem0, gsem1]
    pltpu.sync_copy(zerosF.at[pl.ds(s * RPT, RPT)], agg.at[pl.ds(s * RPT, RPT)])
    pltpu.sync_copy(src2d.at[pl.ds(base, GI)], src_a)
    pltpu.sync_copy(dst2d.at[pl.ds(base, GI)], dst_a)
    plsc.subcore_barrier()

    for G in range(NG):
        sg, dg = idx[G % 2]
        sn, dn = idx[(G + 1) % 2]
        if G + 1 < NG:
            pltpu.async_copy(src2d.at[pl.ds(base + (G + 1) * GI, GI)], sn, isem)
            pltpu.async_copy(dst2d.at[pl.ds(base + (G + 1) * GI, GI)], dn, isem)
        pltpu.async_copy(hs.at[sg.at[0]], rows0, gsem0)
        pltpu.async_copy(hs.at[sg.at[1]], rows1, gsem1)

        def pair(p, carry, sg=sg, dg=dg):
            for b in range(2):
                jj = p * 2 + b
                pltpu.make_async_copy(hs.at[sg.at[jj]], rows[b], gsems[b]).wait()
                pltpu.sync_copy(rows[b], agg.at[dg.at[jj]], add=True)

                @pl.when(jj < GI - 2)
                def _refill():
                    pltpu.async_copy(hs.at[sg.at[jj + 2]], rows[b], gsems[b])

            return carry

        lax.fori_loop(0, GI // 2, pair, 0)
        if G + 1 < NG:
            pltpu.make_async_copy(
                src2d.at[pl.ds(base + (G + 1) * GI, GI)], sn, isem).wait()
            pltpu.make_async_copy(
                dst2d.at[pl.ds(base + (G + 1) * GI, GI)], dn, isem).wait()

    plsc.subcore_barrier()
    pltpu.sync_copy(agg.at[pl.ds(s * RPT, RPT)],
                    out.at[pl.ds(c * NPAD + s * RPT, RPT)])


_edge_call = functools.partial(
    pl.kernel,
    out_type=jax.ShapeDtypeStruct((NC * NPAD, F), jnp.float32),
    mesh=_MESH,
    scratch_types=[
        pltpu.VMEM_SHARED((NPAD, F), jnp.float32),
        pltpu.VMEM((GI, CHUNK), jnp.int32),
        pltpu.VMEM((GI, CHUNK), jnp.int32),
        pltpu.VMEM((GI, CHUNK), jnp.int32),
        pltpu.VMEM((GI, CHUNK), jnp.int32),
        pltpu.VMEM((CHUNK, F), jnp.float32),
        pltpu.VMEM((CHUNK, F), jnp.float32),
        pltpu.SemaphoreType.DMA,
        pltpu.SemaphoreType.DMA,
        pltpu.SemaphoreType.DMA,
    ],
)(_edge_body)


_BPW = B // NW


def _gather_body(y2, idx_hbm, out, idx_v, rows, sem):
    c = lax.axis_index("c")
    s = lax.axis_index("s")
    wid = c * NS + s
    base = wid * _BPW
    pltpu.sync_copy(idx_hbm.at[pl.ds(base, _BPW)], idx_v)
    pltpu.async_copy(y2.at[idx_v], rows, sem).wait()
    pltpu.sync_copy(rows, out.at[pl.ds(base, _BPW)])


_gather_call = functools.partial(
    pl.kernel,
    out_type=jax.ShapeDtypeStruct((B, F), jnp.float32),
    mesh=_MESH,
    scratch_types=[
        pltpu.VMEM((_BPW,), jnp.int32),
        pltpu.VMEM((_BPW, F), jnp.float32),
        pltpu.SemaphoreType.DMA,
    ],
)(_gather_body)


def _mm1_body(x, w, d0, d1, h_o, hs_o, dinv_o):
    h = jnp.dot(x[...], w[...], preferred_element_type=jnp.float32,
                precision=lax.Precision.HIGHEST)
    deg = d0[...] + d1[...] + 1.0
    dinv = lax.rsqrt(jnp.maximum(deg, 1.0))
    h_o[...] = h
    hs_o[...] = dinv * h
    dinv_o[...] = dinv


_mm1_call = pl.pallas_call(
    _mm1_body,
    grid=(GRID,),
    in_specs=[
        pl.BlockSpec((BLK, F), lambda i: (i, 0)),
        pl.BlockSpec((F, F), lambda i: (0, 0)),
        pl.BlockSpec((BLK, 1), lambda i: (i, 0)),
        pl.BlockSpec((BLK, 1), lambda i: (i, 0)),
    ],
    out_specs=[
        pl.BlockSpec((BLK, F), lambda i: (i, 0)),
        pl.BlockSpec((BLK, F), lambda i: (i, 0)),
        pl.BlockSpec((BLK, 1), lambda i: (i, 0)),
    ],
    out_shape=[
        jax.ShapeDtypeStruct((N, F), jnp.float32),
        jax.ShapeDtypeStruct((N, F), jnp.float32),
        jax.ShapeDtypeStruct((N, 1), jnp.float32),
    ],
)


def _mm2_body(x, w, dinv, h_o, hs_o):
    h = jnp.dot(x[...], w[...], preferred_element_type=jnp.float32,
                precision=lax.Precision.HIGHEST)
    h_o[...] = h
    hs_o[...] = dinv[...] * h


_mm2_call = pl.pallas_call(
    _mm2_body,
    grid=(GRID,),
    in_specs=[
        pl.BlockSpec((BLK, F), lambda i: (i, 0)),
        pl.BlockSpec((F, F), lambda i: (0, 0)),
        pl.BlockSpec((BLK, 1), lambda i: (i, 0)),
    ],
    out_specs=[
        pl.BlockSpec((BLK, F), lambda i: (i, 0)),
        pl.BlockSpec((BLK, F), lambda i: (i, 0)),
    ],
    out_shape=[
        jax.ShapeDtypeStruct((N, F), jnp.float32),
        jax.ShapeDtypeStruct((N, F), jnp.float32),
    ],
)


def _stats_body(s0, s1, h, dinv, k2, b, y_o, st_o, acc):
    i = pl.program_id(0)

    @pl.when(i == 0)
    def _init():
        acc[...] = jnp.zeros_like(acc)

    hv = h[...]
    dv = dinv[...]
    agg = dv * (s0[...] + s1[...] + dv * hv)
    k2v = k2[0, 0]
    y = agg - k2v * hv + b[...]
    y_o[...] = y
    r = hv - agg - k2v * hv
    acc[0:1, :] += jnp.sum(y, axis=0, keepdims=True)
    acc[1:2, :] += jnp.sum(y * y, axis=0, keepdims=True)
    acc[2:3, :] += jnp.sum(r * r, axis=0, keepdims=True)

    @pl.when(i == GRID - 1)
    def _fin():
        a = acc[...]
        loss = jnp.sum(a[2:3, :]) * (1.0 / (N * F))
        st_o[...] = jnp.concatenate(
            [a[0:3, :], jnp.full((1, F), loss, jnp.float32),
             jnp.zeros((4, F), jnp.float32)], axis=0)


_stats_call = pl.pallas_call(
    _stats_body,
    grid=(GRID,),
    in_specs=[
        pl.BlockSpec((BLK, F), lambda i: (i, 0)),
        pl.BlockSpec((BLK, F), lambda i: (i, 0)),
        pl.BlockSpec((BLK, F), lambda i: (i, 0)),
        pl.BlockSpec((BLK, 1), lambda i: (i, 0)),
        pl.BlockSpec((1, 1), lambda i: (0, 0)),
        pl.BlockSpec((1, F), lambda i: (0, 0)),
    ],
    out_specs=[
        pl.BlockSpec((BLK, F), lambda i: (i, 0)),
        pl.BlockSpec((8, F), lambda i: (0, 0)),
    ],
    out_shape=[
        jax.ShapeDtypeStruct((N, F), jnp.float32),
        jax.ShapeDtypeStruct((8, F), jnp.float32),
    ],
    scratch_shapes=[pltpu.VMEM((8, F), jnp.float32)],
)


def _bn_coeffs(st, g, be):
    mu = st[0:1, :] * (1.0 / N)
    var = st[1:2, :] * (1.0 / N) - mu * mu
    inv = lax.rsqrt(var + 1e-5)
    return mu, inv * g[...], be[...]


def _apply_body(y, st, g, be, x_o):
    mu, scale, shift = _bn_coeffs(st, g, be)
    x_o[...] = jnp.tanh((y[...] - mu) * scale + shift)


_apply_call = pl.pallas_call(
    _apply_body,
    grid=(GRID,),
    in_specs=[
        pl.BlockSpec((BLK, F), lambda i: (i, 0)),
        pl.BlockSpec((8, F), lambda i: (0, 0)),
        pl.BlockSpec((1, F), lambda i: (0, 0)),
        pl.BlockSpec((1, F), lambda i: (0, 0)),
    ],
    out_specs=pl.BlockSpec((BLK, F), lambda i: (i, 0)),
    out_shape=jax.ShapeDtypeStruct((N, F), jnp.float32),
)


def _final_body(yb, st, g, be, o):
    mu, scale, shift = _bn_coeffs(st, g, be)
    t = jnp.tanh((yb[...] - mu) * scale + shift)
    m = jnp.max(t, axis=1, keepdims=True)
    lse = jnp.log(jnp.sum(jnp.exp(t - m), axis=1, keepdims=True)) + m
    o[...] = t - lse


_final_call = pl.pallas_call(
    _final_body,
    grid=(1,),
    in_specs=[
        pl.BlockSpec((B, F), lambda i: (0, 0)),
        pl.BlockSpec((8, F), lambda i: (0, 0)),
        pl.BlockSpec((1, F), lambda i: (0, 0)),
        pl.BlockSpec((1, F), lambda i: (0, 0)),
    ],
    out_specs=pl.BlockSpec((B, F), lambda i: (0, 0)),
    out_shape=jax.ShapeDtypeStruct((B, F), jnp.float32),
)


def kernel(features, edge_index, batch_nodes, device, W1, b1, k2_1, W2, b2,
           k2_2, g1, be1, g2, be2):
    del device
    src = edge_index[0].astype(jnp.int32)
    dst = edge_index[1].astype(jnp.int32)
    npad = EPAD - E
    src2d = jnp.concatenate([src, jnp.zeros((npad,), jnp.int32)]).reshape(
        ECHUNKS, CHUNK)
    dst2d = jnp.concatenate([dst, jnp.full((npad,), N, jnp.int32)]).reshape(
        ECHUNKS, CHUNK)
    zerosF = jnp.zeros((NPAD, F), jnp.float32)
    onesF = jnp.ones((CHUNK, F), jnp.float32)

    deg_part = _deg_call(dst2d, zerosF, onesF)
    deg0 = deg_part[0:N, 0:1]
    deg1 = deg_part[NPAD:NPAD + N, 0:1]

    h1, hs1, dinv = _mm1_call(features, W1, deg0, deg1)
    s1p = _edge_call(hs1, src2d, dst2d, zerosF)
    y1, st1 = _stats_call(s1p[0:N], s1p[NPAD:NPAD + N], h1, dinv,
                          k2_1.reshape(1, 1), b1.reshape(1, F))
    x2 = _apply_call(y1, st1, g1.reshape(1, F), be1.reshape(1, F))

    h2, hs2 = _mm2_call(x2, W2, dinv)
    s2p = _edge_call(hs2, src2d, dst2d, zerosF)
    y2, st2 = _stats_call(s2p[0:N], s2p[NPAD:NPAD + N], h2, dinv,
                          k2_2.reshape(1, 1), b2.reshape(1, F))

    yb = _gather_call(y2, batch_nodes.astype(jnp.int32))
    logp = _final_call(yb, st2, g2.reshape(1, F), be2.reshape(1, F))
    return logp, st1[3, 0]


# speedup vs baseline: 9.4205x; 1.1206x over previous
import functools

import jax
import jax.numpy as jnp
from jax import lax
from jax.experimental import pallas as pl
from jax.experimental.pallas import tpu as pltpu
from jax.experimental.pallas import tpu_sc as plsc

N = 10000
E = 320000
F = 128
B = 1024

NC, NS = 2, 16
NW = NC * NS
NPAD = 10112
RPT = NPAD // NS

CHUNK = 128
NCHUNK = 80
EPW = NCHUNK * CHUNK
EPAD = EPW * NW
ECHUNKS = EPAD // CHUNK

BLK = 1000
GRID = N // BLK

_MESH = plsc.VectorSubcoreMesh(
    core_axis_name="c", subcore_axis_name="s", num_cores=NC, num_subcores=NS)


_DEG_GRP = 8


def _deg_body(dst2d, zerosF, onesF, out, table, ones_v, idx_v, sem):
    c = lax.axis_index("c")
    s = lax.axis_index("s")
    wid = c * NS + s
    pltpu.sync_copy(zerosF.at[pl.ds(s * RPT, RPT)], table.at[pl.ds(s * RPT, RPT)])
    pltpu.sync_copy(onesF, ones_v)
    pltpu.sync_copy(dst2d.at[pl.ds(wid * NCHUNK, NCHUNK)], idx_v)
    plsc.subcore_barrier()

    def group(g, carry):
        for b in range(_DEG_GRP):
            pltpu.async_copy(ones_v, table.at[idx_v.at[g * _DEG_GRP + b]], sem,
                             add=True)
        for b in range(_DEG_GRP):
            pltpu.make_async_copy(ones_v, table.at[idx_v.at[g * _DEG_GRP + b]],
                                  sem).wait()
        return carry

    lax.fori_loop(0, NCHUNK // _DEG_GRP, group, 0)
    plsc.subcore_barrier()
    pltpu.sync_copy(table.at[pl.ds(s * RPT, RPT)],
                    out.at[pl.ds(c * NPAD + s * RPT, RPT)])


_deg_call = functools.partial(
    pl.kernel,
    out_type=jax.ShapeDtypeStruct((NC * NPAD, F), jnp.float32),
    mesh=_MESH,
    scratch_types=[
        pltpu.VMEM_SHARED((NPAD, F), jnp.float32),
        pltpu.VMEM((CHUNK, F), jnp.float32),
        pltpu.VMEM((NCHUNK, CHUNK), jnp.int32),
        pltpu.SemaphoreType.DMA,
    ],
)(_deg_body)


GI = 16
NG = NCHUNK // GI


def _edge_body(hs, src2d, dst2d, zerosF, out, agg,
               src_a, dst_a, src_b, dst_b, rows0, rows1, gsem0, gsem1, isem):
    c = lax.axis_index("c")
    s = lax.axis_index("s")
    wid = c * NS + s
    base = wid * NCHUNK
    idx = [(src_a, dst_a), (src_b, dst_b)]
    rows = [rows0, rows1]
    gsems = [gsem0, gsem1]
    pltpu.sync_copy(zerosF.at[pl.ds(s * RPT, RPT)], agg.at[pl.ds(s * RPT, RPT)])
    pltpu.sync_copy(src2d.at[pl.ds(base, GI)], src_a)
    pltpu.sync_copy(dst2d.at[pl.ds(base, GI)], dst_a)
    plsc.subcore_barrier()

    for G in range(NG):
        sg, dg = idx[G % 2]
        sn, dn = idx[(G + 1) % 2]
        if G + 1 < NG:
            pltpu.async_copy(src2d.at[pl.ds(base + (G + 1) * GI, GI)], sn, isem)
            pltpu.async_copy(dst2d.at[pl.ds(base + (G + 1) * GI, GI)], dn, isem)
        pltpu.async_copy(hs.at[sg.at[0]], rows0, gsem0)
        pltpu.async_copy(hs.at[sg.at[1]], rows1, gsem1)

        def pair(p, carry, sg=sg, dg=dg):
            for b in range(2):
                jj = p * 2 + b
                pltpu.make_async_copy(hs.at[sg.at[jj]], rows[b], gsems[b]).wait()
                pltpu.sync_copy(rows[b], agg.at[dg.at[jj]], add=True)

                @pl.when(jj < GI - 2)
                def _refill():
                    pltpu.async_copy(hs.at[sg.at[jj + 2]], rows[b], gsems[b])

            return carry

        lax.fori_loop(0, GI // 2, pair, 0)
        if G + 1 < NG:
            pltpu.make_async_copy(
                src2d.at[pl.ds(base + (G + 1) * GI, GI)], sn, isem).wait()
            pltpu.make_async_copy(
                dst2d.at[pl.ds(base + (G + 1) * GI, GI)], dn, isem).wait()

    plsc.subcore_barrier()
    pltpu.sync_copy(agg.at[pl.ds(s * RPT, RPT)],
                    out.at[pl.ds(c * NPAD + s * RPT, RPT)])


_edge_call = functools.partial(
    pl.kernel,
    out_type=jax.ShapeDtypeStruct((NC * NPAD, F), jnp.float32),
    mesh=_MESH,
    scratch_types=[
        pltpu.VMEM_SHARED((NPAD, F), jnp.float32),
        pltpu.VMEM((GI, CHUNK), jnp.int32),
        pltpu.VMEM((GI, CHUNK), jnp.int32),
        pltpu.VMEM((GI, CHUNK), jnp.int32),
        pltpu.VMEM((GI, CHUNK), jnp.int32),
        pltpu.VMEM((CHUNK, F), jnp.float32),
        pltpu.VMEM((CHUNK, F), jnp.float32),
        pltpu.SemaphoreType.DMA,
        pltpu.SemaphoreType.DMA,
        pltpu.SemaphoreType.DMA,
    ],
)(_edge_body)


_BPW = B // NW


def _gather_body(y2, idx_hbm, out, idx_v, rows, sem):
    c = lax.axis_index("c")
    s = lax.axis_index("s")
    wid = c * NS + s
    base = wid * _BPW
    pltpu.sync_copy(idx_hbm.at[pl.ds(base, _BPW)], idx_v)
    pltpu.async_copy(y2.at[idx_v], rows, sem).wait()
    pltpu.sync_copy(rows, out.at[pl.ds(base, _BPW)])


_gather_call = functools.partial(
    pl.kernel,
    out_type=jax.ShapeDtypeStruct((B, F), jnp.float32),
    mesh=_MESH,
    scratch_types=[
        pltpu.VMEM((_BPW,), jnp.int32),
        pltpu.VMEM((_BPW, F), jnp.float32),
        pltpu.SemaphoreType.DMA,
    ],
)(_gather_body)


def _mm1_body(x, w, d0, d1, h_o, hs_o, dinv_o):
    h = jnp.dot(x[...], w[...], preferred_element_type=jnp.float32,
                precision=lax.Precision.HIGHEST)
    deg = d0[...] + d1[...] + 1.0
    dinv = lax.rsqrt(jnp.maximum(deg, 1.0))
    h_o[...] = h
    hs_o[...] = dinv * h
    dinv_o[...] = dinv


_mm1_call = pl.pallas_call(
    _mm1_body,
    grid=(GRID,),
    in_specs=[
        pl.BlockSpec((BLK, F), lambda i: (i, 0)),
        pl.BlockSpec((F, F), lambda i: (0, 0)),
        pl.BlockSpec((BLK, 1), lambda i: (i, 0)),
        pl.BlockSpec((BLK, 1), lambda i: (i, 0)),
    ],
    out_specs=[
        pl.BlockSpec((BLK, F), lambda i: (i, 0)),
        pl.BlockSpec((BLK, F), lambda i: (i, 0)),
        pl.BlockSpec((BLK, 1), lambda i: (i, 0)),
    ],
    out_shape=[
        jax.ShapeDtypeStruct((N, F), jnp.float32),
        jax.ShapeDtypeStruct((N, F), jnp.float32),
        jax.ShapeDtypeStruct((N, 1), jnp.float32),
    ],
)


def _mm2_body(x, w, dinv, h_o, hs_o):
    h = jnp.dot(x[...], w[...], preferred_element_type=jnp.float32,
                precision=lax.Precision.HIGHEST)
    h_o[...] = h
    hs_o[...] = dinv[...] * h


_mm2_call = pl.pallas_call(
    _mm2_body,
    grid=(GRID,),
    in_specs=[
        pl.BlockSpec((BLK, F), lambda i: (i, 0)),
        pl.BlockSpec((F, F), lambda i: (0, 0)),
        pl.BlockSpec((BLK, 1), lambda i: (i, 0)),
    ],
    out_specs=[
        pl.BlockSpec((BLK, F), lambda i: (i, 0)),
        pl.BlockSpec((BLK, F), lambda i: (i, 0)),
    ],
    out_shape=[
        jax.ShapeDtypeStruct((N, F), jnp.float32),
        jax.ShapeDtypeStruct((N, F), jnp.float32),
    ],
)


def _stats_body(s0, s1, h, dinv, k2, b, y_o, st_o, acc):
    i = pl.program_id(0)

    @pl.when(i == 0)
    def _init():
        acc[...] = jnp.zeros_like(acc)

    hv = h[...]
    dv = dinv[...]
    agg = dv * (s0[...] + s1[...] + dv * hv)
    k2v = k2[0, 0]
    y = agg - k2v * hv + b[...]
    y_o[...] = y
    r = hv - agg - k2v * hv
    acc[0:1, :] += jnp.sum(y, axis=0, keepdims=True)
    acc[1:2, :] += jnp.sum(y * y, axis=0, keepdims=True)
    acc[2:3, :] += jnp.sum(r * r, axis=0, keepdims=True)

    @pl.when(i == GRID - 1)
    def _fin():
        a = acc[...]
        loss = jnp.sum(a[2:3, :]) * (1.0 / (N * F))
        st_o[...] = jnp.concatenate(
            [a[0:3, :], jnp.full((1, F), loss, jnp.float32),
             jnp.zeros((4, F), jnp.float32)], axis=0)


_stats_call = pl.pallas_call(
    _stats_body,
    grid=(GRID,),
    in_specs=[
        pl.BlockSpec((BLK, F), lambda i: (i, 0)),
        pl.BlockSpec((BLK, F), lambda i: (i, 0)),
        pl.BlockSpec((BLK, F), lambda i: (i, 0)),
        pl.BlockSpec((BLK, 1), lambda i: (i, 0)),
        pl.BlockSpec((1, 1), lambda i: (0, 0)),
        pl.BlockSpec((1, F), lambda i: (0, 0)),
    ],
    out_specs=[
        pl.BlockSpec((BLK, F), lambda i: (i, 0)),
        pl.BlockSpec((8, F), lambda i: (0, 0)),
    ],
    out_shape=[
        jax.ShapeDtypeStruct((N, F), jnp.float32),
        jax.ShapeDtypeStruct((8, F), jnp.float32),
    ],
    scratch_shapes=[pltpu.VMEM((8, F), jnp.float32)],
)


def _bn_coeffs(st, g, be):
    mu = st[0:1, :] * (1.0 / N)
    var = st[1:2, :] * (1.0 / N) - mu * mu
    inv = lax.rsqrt(var + 1e-5)
    return mu, inv * g[...], be[...]


def _apply_body(y, st, g, be, x_o):
    mu, scale, shift = _bn_coeffs(st, g, be)
    x_o[...] = jnp.tanh((y[...] - mu) * scale + shift)


_apply_call = pl.pallas_call(
    _apply_body,
    grid=(GRID,),
    in_specs=[
        pl.BlockSpec((BLK, F), lambda i: (i, 0)),
        pl.BlockSpec((8, F), lambda i: (0, 0)),
        pl.BlockSpec((1, F), lambda i: (0, 0)),
        pl.BlockSpec((1, F), lambda i: (0, 0)),
    ],
    out_specs=pl.BlockSpec((BLK, F), lambda i: (i, 0)),
    out_shape=jax.ShapeDtypeStruct((N, F), jnp.float32),
)


def _final_body(yb, st, g, be, o):
    mu, scale, shift = _bn_coeffs(st, g, be)
    t = jnp.tanh((yb[...] - mu) * scale + shift)
    m = jnp.max(t, axis=1, keepdims=True)
    lse = jnp.log(jnp.sum(jnp.exp(t - m), axis=1, keepdims=True)) + m
    o[...] = t - lse


_final_call = pl.pallas_call(
    _final_body,
    grid=(1,),
    in_specs=[
        pl.BlockSpec((B, F), lambda i: (0, 0)),
        pl.BlockSpec((8, F), lambda i: (0, 0)),
        pl.BlockSpec((1, F), lambda i: (0, 0)),
        pl.BlockSpec((1, F), lambda i: (0, 0)),
    ],
    out_specs=pl.BlockSpec((B, F), lambda i: (0, 0)),
    out_shape=jax.ShapeDtypeStruct((B, F), jnp.float32),
)


def kernel(features, edge_index, batch_nodes, device, W1, b1, k2_1, W2, b2,
           k2_2, g1, be1, g2, be2):
    del device
    src = edge_index[0].astype(jnp.int32)
    dst = edge_index[1].astype(jnp.int32)
    ppw = (EPAD - E) // NW
    pad_src = jnp.zeros((NW, ppw), jnp.int32)
    pad_dst = jnp.broadcast_to(N + (jnp.arange(ppw, dtype=jnp.int32)
                                    % (NPAD - N)), (NW, ppw))
    src2d = jnp.concatenate([src.reshape(NW, E // NW), pad_src],
                            axis=1).reshape(ECHUNKS, CHUNK)
    dst2d = jnp.concatenate([dst.reshape(NW, E // NW), pad_dst],
                            axis=1).reshape(ECHUNKS, CHUNK)
    zerosF = jnp.zeros((NPAD, F), jnp.float32)
    onesF = jnp.ones((CHUNK, F), jnp.float32)

    deg_part = _deg_call(dst2d, zerosF, onesF)
    deg0 = deg_part[0:N, 0:1]
    deg1 = deg_part[NPAD:NPAD + N, 0:1]

    h1, hs1, dinv = _mm1_call(features, W1, deg0, deg1)
    s1p = _edge_call(hs1, src2d, dst2d, zerosF)
    y1, st1 = _stats_call(s1p[0:N], s1p[NPAD:NPAD + N], h1, dinv,
                          k2_1.reshape(1, 1), b1.reshape(1, F))
    x2 = _apply_call(y1, st1, g1.reshape(1, F), be1.reshape(1, F))

    h2, hs2 = _mm2_call(x2, W2, dinv)
    s2p = _edge_call(hs2, src2d, dst2d, zerosF)
    y2, st2 = _stats_call(s2p[0:N], s2p[NPAD:NPAD + N], h2, dinv,
                          k2_2.reshape(1, 1), b2.reshape(1, F))

    yb = _gather_call(y2, batch_nodes.astype(jnp.int32))
    logp = _final_call(yb, st2, g2.reshape(1, F), be2.reshape(1, F))
    return logp, st1[3, 0]
